# baseline (speedup 1.0000x reference)
import functools

import jax
import jax.numpy as jnp
from jax import lax
from jax.experimental import pallas as pl
from jax.experimental.pallas import tpu as pltpu
from jax.experimental.pallas import tpu_sc as plsc

D_MODEL = 1024
D4 = D_MODEL // 4
NC, NS, L = 2, 16, 16
NW = NC * NS
SUB = 16
NBUF = 4
MIN_PER_HOUR = 60
MIN_PER_DAY = 60 * 24
MIN_PER_MONTH = 60 * 24 * 32


def _fill_segmented(buf, tbl, col, period, nrows, s0):
    row0 = lax.rem(lax.div(s0, period), nrows)
    row1 = lax.rem(row0 + 1, nrows)
    split = lax.min(period - lax.rem(s0, period), SUB)
    vals0 = [tbl[row0, pl.ds(c * L, L)] for c in range(D4 // L)]
    vals1 = [tbl[row1, pl.ds(c * L, L)] for c in range(D4 // L)]

    def store0(p, _):
        for c in range(D4 // L):
            buf[p, pl.ds(col + c * L, L)] = vals0[c]
        return _

    def store1(p, _):
        for c in range(D4 // L):
            buf[p, pl.ds(col + c * L, L)] = vals1[c]
        return _

    lax.fori_loop(0, split, store0, None)
    lax.fori_loop(split, SUB, store1, None)


def _sc_body(chunk, batch, minute_hbm, hour_hbm, day_hbm, month_hbm, out_hbm,
             tb_m, tb_h, tb_d, tb_mo, rows0, rows1, rows2, rows3,
             sem_s, sem_w):
    wid = lax.axis_index("s") * NC + lax.axis_index("c")
    base = wid * chunk

    stages = [pltpu.async_copy(src, dst, sem_s) for src, dst in
              ((minute_hbm, tb_m), (hour_hbm, tb_h),
               (day_hbm, tb_d), (month_hbm, tb_mo))]
    for s in stages:
        s.wait()

    def _row_pair(period, nrows, s0):
        row0 = lax.rem(lax.div(s0, period), nrows)
        row1 = lax.rem(row0 + 1, nrows)
        split = lax.min(period - lax.rem(s0, period), SUB)
        return row0, row1, split

    def assemble(k, buf):
        s0 = base + k * SUB
        m0 = lax.rem(s0, MIN_PER_HOUR)
        hrow0, hrow1, hsplit = _row_pair(MIN_PER_HOUR, 24, s0)
        drow0, drow1, dsplit = _row_pair(MIN_PER_DAY, 32, s0)
        if MIN_PER_MONTH % chunk != 0:
            _fill_segmented(buf, tb_mo, 3 * D4, MIN_PER_MONTH, 13, s0)

        @plsc.parallel_loop(0, SUB, unroll=2)
        def _fill(p):
            r = m0 + p
            mrow = lax.select(r >= MIN_PER_HOUR, r - MIN_PER_HOUR, r)
            hrow = lax.select(p < hsplit, hrow0, hrow1)
            drow = lax.select(p < dsplit, drow0, drow1)
            for c in range(D4 // L):
                buf[p, pl.ds(c * L, L)] = tb_m[mrow, pl.ds(c * L, L)]
                buf[p, pl.ds(D4 + c * L, L)] = tb_h[hrow, pl.ds(c * L, L)]
                buf[p, pl.ds(2 * D4 + c * L, L)] = \
                    tb_d[drow, pl.ds(c * L, L)]

    bufs = (rows0, rows1, rows2, rows3)
    nsub = chunk // SUB

    if MIN_PER_MONTH % chunk == 0:
        for rb in bufs:
            _fill_segmented(rb, tb_mo, 3 * D4, MIN_PER_MONTH, 13, base)
    def fire_writes(k, buf):
        for b in range(batch):
            pltpu.async_copy(
                buf, out_hbm.at[b, pl.ds(base + k * SUB, SUB)], sem_w)

    def drain_writes(buf):
        for _ in range(batch):
            pltpu.make_async_copy(
                out_hbm.at[0, pl.ds(base, SUB)], buf, sem_w).wait()

    for k in range(NBUF):
        assemble(k, bufs[k])
        fire_writes(k, bufs[k])

    @pl.loop(1, nsub // NBUF)
    def _group(g):
        for i in range(NBUF):
            k = g * NBUF + i
            drain_writes(bufs[i])
            assemble(k, bufs[i])
            fire_writes(k, bufs[i])

    for i in range(NBUF):
        drain_writes(bufs[i])


def kernel(x, minute_table, hour_table, day_table, month_table):
    batch, seq_len, _ = x.shape
    chunk = seq_len // NW
    mesh = plsc.VectorSubcoreMesh(core_axis_name="c", subcore_axis_name="s",
                                  num_cores=NC, num_subcores=NS)

    run = pl.kernel(
        functools.partial(_sc_body, chunk, batch),
        out_type=jax.ShapeDtypeStruct((batch, seq_len, D_MODEL), jnp.float32),
        mesh=mesh,
        scratch_types=[
            pltpu.VMEM((60, D4), jnp.float32),
            pltpu.VMEM((24, D4), jnp.float32),
            pltpu.VMEM((32, D4), jnp.float32),
            pltpu.VMEM((13, D4), jnp.float32),
            pltpu.VMEM((SUB, D_MODEL), jnp.float32),
            pltpu.VMEM((SUB, D_MODEL), jnp.float32),
            pltpu.VMEM((SUB, D_MODEL), jnp.float32),
            pltpu.VMEM((SUB, D_MODEL), jnp.float32),
            pltpu.SemaphoreType.DMA,
            pltpu.SemaphoreType.DMA,
        ],
    )
    return run(minute_table, hour_table, day_table, month_table)


# speedup vs baseline: 1.0278x; 1.0278x over previous
import functools

import jax
import jax.numpy as jnp
from jax import lax
from jax.experimental import pallas as pl
from jax.experimental.pallas import tpu as pltpu
from jax.experimental.pallas import tpu_sc as plsc

D_MODEL = 1024
D4 = D_MODEL // 4
NC, NS, L = 2, 16, 16
NW = NC * NS
SUB = 16
NBUF = 4
MIN_PER_HOUR = 60
MIN_PER_DAY = 60 * 24
MIN_PER_MONTH = 60 * 24 * 32


def _fill_segmented(buf, tbl, col, period, nrows, s0):
    row0 = lax.rem(lax.div(s0, period), nrows)
    row1 = lax.rem(row0 + 1, nrows)
    split = lax.min(period - lax.rem(s0, period), SUB)
    vals0 = [tbl[row0, pl.ds(c * L, L)] for c in range(D4 // L)]
    vals1 = [tbl[row1, pl.ds(c * L, L)] for c in range(D4 // L)]

    def store0(p, _):
        for c in range(D4 // L):
            buf[p, pl.ds(col + c * L, L)] = vals0[c]
        return _

    def store1(p, _):
        for c in range(D4 // L):
            buf[p, pl.ds(col + c * L, L)] = vals1[c]
        return _

    lax.fori_loop(0, split, store0, None)
    lax.fori_loop(split, SUB, store1, None)


def _sc_body(chunk, batch, minute_hbm, hour_hbm, day_hbm, month_hbm, out_hbm,
             tb_m, tb_h, tb_d, tb_mo, rows0, rows1, rows2, rows3,
             sem_s, sem_w):
    wid = lax.axis_index("s") * NC + lax.axis_index("c")
    base = wid * chunk

    stages = [pltpu.async_copy(src, dst, sem_s) for src, dst in
              ((minute_hbm, tb_m), (hour_hbm, tb_h),
               (day_hbm, tb_d), (month_hbm, tb_mo))]
    for s in stages:
        s.wait()

    def _row_pair(period, nrows, s0):
        row0 = lax.rem(lax.div(s0, period), nrows)
        row1 = lax.rem(row0 + 1, nrows)
        split = lax.min(period - lax.rem(s0, period), SUB)
        return row0, row1, split

    def assemble(k, buf):
        s0 = base + k * SUB
        m0 = lax.rem(s0, MIN_PER_HOUR)
        hrow0, hrow1, hsplit = _row_pair(MIN_PER_HOUR, 24, s0)
        drow0, drow1, dsplit = _row_pair(MIN_PER_DAY, 32, s0)
        if MIN_PER_MONTH % chunk != 0:
            _fill_segmented(buf, tb_mo, 3 * D4, MIN_PER_MONTH, 13, s0)

        hv1 = [tb_h[hrow1, pl.ds(c * L, L)] for c in range(D4 // L)]
        hdiff = [tb_h[hrow0, pl.ds(c * L, L)] - hv1[c]
                 for c in range(D4 // L)]
        dv1 = [tb_d[drow1, pl.ds(c * L, L)] for c in range(D4 // L)]
        ddiff = [tb_d[drow0, pl.ds(c * L, L)] - dv1[c]
                 for c in range(D4 // L)]

        @plsc.parallel_loop(0, SUB, unroll=2)
        def _fill(p):
            r = m0 + p
            mrow = lax.select(r >= MIN_PER_HOUR, r - MIN_PER_HOUR, r)
            hm = lax.select(p < hsplit, jnp.float32(1), jnp.float32(0))
            dm = lax.select(p < dsplit, jnp.float32(1), jnp.float32(0))
            hmv = jax.lax.broadcast_in_dim(hm, (L,), ())
            dmv = jax.lax.broadcast_in_dim(dm, (L,), ())
            for c in range(D4 // L):
                buf[p, pl.ds(c * L, L)] = tb_m[mrow, pl.ds(c * L, L)]
                buf[p, pl.ds(D4 + c * L, L)] = hv1[c] + hdiff[c] * hmv
                buf[p, pl.ds(2 * D4 + c * L, L)] = dv1[c] + ddiff[c] * dmv

    bufs = (rows0, rows1, rows2, rows3)
    nsub = chunk // SUB

    if MIN_PER_MONTH % chunk == 0:
        for rb in bufs:
            _fill_segmented(rb, tb_mo, 3 * D4, MIN_PER_MONTH, 13, base)
    def fire_writes(k, buf):
        for b in range(batch):
            pltpu.async_copy(
                buf, out_hbm.at[b, pl.ds(base + k * SUB, SUB)], sem_w)

    def drain_writes(buf):
        for _ in range(batch):
            pltpu.make_async_copy(
                out_hbm.at[0, pl.ds(base, SUB)], buf, sem_w).wait()

    for k in range(NBUF):
        assemble(k, bufs[k])
        fire_writes(k, bufs[k])

    @pl.loop(1, nsub // NBUF)
    def _group(g):
        for i in range(NBUF):
            k = g * NBUF + i
            drain_writes(bufs[i])
            assemble(k, bufs[i])
            fire_writes(k, bufs[i])

    for i in range(NBUF):
        drain_writes(bufs[i])


def kernel(x, minute_table, hour_table, day_table, month_table):
    batch, seq_len, _ = x.shape
    chunk = seq_len // NW
    mesh = plsc.VectorSubcoreMesh(core_axis_name="c", subcore_axis_name="s",
                                  num_cores=NC, num_subcores=NS)

    run = pl.kernel(
        functools.partial(_sc_body, chunk, batch),
        out_type=jax.ShapeDtypeStruct((batch, seq_len, D_MODEL), jnp.float32),
        mesh=mesh,
        scratch_types=[
            pltpu.VMEM((60, D4), jnp.float32),
            pltpu.VMEM((24, D4), jnp.float32),
            pltpu.VMEM((32, D4), jnp.float32),
            pltpu.VMEM((13, D4), jnp.float32),
            pltpu.VMEM((SUB, D_MODEL), jnp.float32),
            pltpu.VMEM((SUB, D_MODEL), jnp.float32),
            pltpu.VMEM((SUB, D_MODEL), jnp.float32),
            pltpu.VMEM((SUB, D_MODEL), jnp.float32),
            pltpu.SemaphoreType.DMA,
            pltpu.SemaphoreType.DMA,
        ],
    )
    return run(minute_table, hour_table, day_table, month_table)
